# baseline (speedup 1.0000x reference)
import functools

import jax
import jax.numpy as jnp
from jax.experimental import pallas as pl
from jax.experimental.pallas import tpu as pltpu
from jax.experimental.pallas import tpu_sc as plsc

_P, _C = 262144, 21
_R, _L = 2048, 128
_CH = 16


def _iota(shape, dim):
    return jax.lax.broadcasted_iota(jnp.int32, shape, dim)


def _keys_body(xt_ref, lab_ref, out_ref):
    x = xt_ref[...]
    lab = lab_ref[...]
    m = jnp.max(x, axis=0)
    lse = m + jnp.log(jnp.sum(jnp.exp(x - m[None]), axis=0))
    for c in range(_C):
        fg = lab == c
        e = lse - x[c] + fg.astype(jnp.float32)
        bits = jax.lax.bitcast_convert_type(e, jnp.int32)
        out_ref[c] = jnp.bitwise_or(
            jnp.bitwise_and(bits, -2), fg.astype(jnp.int32))


def _ce_sub(X, s, dirm):
    rb = jnp.bitwise_and(_iota(X.shape, 0), s) != 0
    Pv = jnp.where(rb, pltpu.roll(X, s, 0), pltpu.roll(X, X.shape[0] - s, 0))
    take_max = dirm == jnp.logical_not(rb)
    return jnp.where(take_max, jnp.maximum(X, Pv), jnp.minimum(X, Pv))


def _ce_lane(X, s, dirm):
    lb = jnp.bitwise_and(_iota(X.shape, 1), s) != 0
    Pv = jnp.where(lb, pltpu.roll(X, s, 1), pltpu.roll(X, X.shape[1] - s, 1))
    take_max = dirm == jnp.logical_not(lb)
    return jnp.where(take_max, jnp.maximum(X, Pv), jnp.minimum(X, Pv))


def _sort_body(in_ref, out_ref, S):
    S[...] = in_ref[0]
    riota = _iota((128, _L), 0)
    liota = _iota((128, _L), 1)

    def p0_chunk(c, _):
        X = S[pl.ds(c * 128, 128), :]
        for k in range(1, 8):
            if k < 7:
                dirm = jnp.bitwise_and(riota, 1 << k) == 0
            else:
                dirm = jnp.bitwise_and(c, 1) == 0
            for j in range(k - 1, -1, -1):
                X = _ce_sub(X, 1 << j, dirm)
        S[pl.ds(c * 128, 128), :] = X
        return 0

    jax.lax.fori_loop(0, _CH, p0_chunk, 0, unroll=False)

    for k in range(8, 19):
        size = 1 << k

        if k >= 12:
            def lane_chunk(c, _, k=k):
                X = S[pl.ds(c * 128, 128), :]
                dirm = jnp.bitwise_and(liota, 1 << (k - 11)) == 0
                for j in range(k - 1, 10, -1):
                    X = _ce_lane(X, 1 << (j - 11), dirm)
                S[pl.ds(c * 128, 128), :] = X
                return 0

            jax.lax.fori_loop(0, _CH, lane_chunk, 0, unroll=False)

        jmax = min(k - 1, 10)
        if jmax >= 7:
            def b1_j(a, _, k=k, size=size, jmax=jmax):
                j = jmax - a
                mm = jnp.left_shift(jnp.int32(1), j - 7)

                def b1_q(q, _):
                    lo = jnp.bitwise_or(
                        jnp.left_shift(jnp.bitwise_and(q, -mm), 1),
                        jnp.bitwise_and(q, mm - 1))
                    A = S[pl.ds(lo * 128, 128), :]
                    B = S[pl.ds((lo + mm) * 128, 128), :]
                    i_low = liota * _R + lo * 128 + riota
                    dirm = jnp.bitwise_and(i_low, size) == 0
                    mn = jnp.minimum(A, B)
                    mx = jnp.maximum(A, B)
                    S[pl.ds(lo * 128, 128), :] = jnp.where(dirm, mx, mn)
                    S[pl.ds((lo + mm) * 128, 128), :] = jnp.where(dirm, mn, mx)
                    return 0

                jax.lax.fori_loop(0, 8, b1_q, 0, unroll=False)
                return 0

            jax.lax.fori_loop(0, jmax - 6, b1_j, 0, unroll=False)

        def b2_chunk(c, _, size=size):
            X = S[pl.ds(c * 128, 128), :]
            i0 = liota * _R + c * 128 + riota
            dirm = jnp.bitwise_and(i0, size) == 0
            for j in range(6, -1, -1):
                X = _ce_sub(X, 1 << j, dirm)
            S[pl.ds(c * 128, 128), :] = X
            return 0

        jax.lax.fori_loop(0, _CH, b2_chunk, 0, unroll=False)

    out_ref[0] = S[...]


_NT = 16
_CHK = _P // _NT
_NB = 256
_FLAT = _C * _P


def _digit(k, shift):
    return jnp.bitwise_and(
        jax.lax.shift_right_logical(jnp.bitwise_not(k), shift), _NB - 1)


def _sc_sort_build():
    mesh = plsc.VectorSubcoreMesh(core_axis_name="c", subcore_axis_name="s")

    @functools.partial(
        pl.kernel,
        out_type=[
            jax.ShapeDtypeStruct((_FLAT,), jnp.int32),
            jax.ShapeDtypeStruct((_FLAT,), jnp.int32),
        ],
        mesh=mesh,
        compiler_params=pltpu.CompilerParams(needs_layout_passes=False),
        scratch_types=[
            pltpu.VMEM((_CHK,), jnp.int32),
            pltpu.VMEM((_CHK,), jnp.int32),
            pltpu.VMEM((_CHK,), jnp.int32),
            pltpu.VMEM((_NB * 16,), jnp.int32),
            pltpu.VMEM((_NB,), jnp.int32),
            pltpu.VMEM((_NT, _NB), jnp.int32),
            pltpu.VMEM((_NB,), jnp.int32),
            pltpu.VMEM_SHARED((_NT, _NB), jnp.int32),
            pltpu.SemaphoreType.DMA,
        ],
    )
    def _sc_sort(keys_hbm, ping_hbm, pong_hbm, in_v, vals_v, idx_v, hist_v,
                 tiletot_v, grid_v, basecv_v, grid_sp, sem):
        core = jax.lax.axis_index("c")
        tid = jax.lax.axis_index("s")
        lane = jax.lax.broadcasted_iota(jnp.int32, (16,), 0)
        ones16 = jnp.ones((16,), jnp.int32)
        zeros16 = jnp.zeros((16,), jnp.int32)

        def per_class(ci, _):
            c = core * 11 + ci

            @pl.when(c < _C)
            def _():
                cbase = c * _P
                tbase = cbase + tid * _CHK

                for p, (src, dst) in enumerate(
                        [(keys_hbm, ping_hbm), (ping_hbm, pong_hbm),
                         (pong_hbm, ping_hbm), (ping_hbm, pong_hbm)]):
                    shift = p * 8
                    pltpu.sync_copy(src.at[pl.ds(tbase, _CHK)], in_v)

                    def zz(b, _):
                        hist_v[pl.ds(b * 16, 16)] = zeros16
                        return 0

                    jax.lax.fori_loop(0, _NB, zz, 0, unroll=8)

                    def hh(v, _):
                        k = plsc.load_gather(in_v, [lane * 1024 + v])
                        d = _digit(k, shift)
                        plsc.addupdate_scatter(hist_v, [d * 16 + lane], ones16)
                        return 0

                    jax.lax.fori_loop(0, 1024, hh, 0, unroll=8)

                    def tt(g, _):
                        acc = zeros16
                        for j in range(16):
                            acc = acc + plsc.load_gather(
                                hist_v, [(g * 16 + lane) * 16 + j])
                        tiletot_v[pl.ds(g * 16, 16)] = acc
                        return 0

                    jax.lax.fori_loop(0, _NT, tt, 0)

                    pltpu.sync_copy(tiletot_v, grid_sp.at[tid])
                    plsc.subcore_barrier()
                    pltpu.sync_copy(grid_sp, grid_v)

                    def scan_g(g, carry):
                        tv = zeros16
                        cv = zeros16
                        for t in range(_NT):
                            row = grid_v[t, pl.ds(g * 16, 16)]
                            tv = tv + row
                            cv = cv + jnp.where(t < tid, row, 0)
                        basecv_v[pl.ds(g * 16, 16)] = (
                            plsc.cumsum(tv) - tv + carry + cv)
                        return carry + jnp.sum(tv)

                    jax.lax.fori_loop(0, _NT, scan_g, cbase)

                    def cc(b, _):
                        bvec = plsc.load_gather(basecv_v, [zeros16 + b])
                        lh = hist_v[pl.ds(b * 16, 16)]
                        hist_v[pl.ds(b * 16, 16)] = (
                            bvec + plsc.cumsum(lh) - lh)
                        return 0

                    jax.lax.fori_loop(0, _NB, cc, 0, unroll=4)

                    def rr(v, _):
                        j16 = lane * 1024 + v
                        k = plsc.load_gather(in_v, [j16])
                        d = _digit(k, shift)
                        cidx = d * 16 + lane
                        pos = plsc.load_gather(hist_v, [cidx])
                        plsc.store_scatter(hist_v, [cidx], pos + 1)
                        plsc.store_scatter(vals_v, [j16], k)
                        plsc.store_scatter(idx_v, [j16], pos)
                        return 0

                    jax.lax.fori_loop(0, 1024, rr, 0, unroll=8)

                    pltpu.async_copy(vals_v, dst.at[idx_v], sem).wait()
                    plsc.subcore_barrier()

            return 0

        jax.lax.fori_loop(0, 11, per_class, 0)

    return _sc_sort


def _loss_body(in_ref, loss_ref, g_ref):
    def cs(t, acc):
        u = in_ref[0, pl.ds(t * 256, 256), :]
        return acc + jnp.sum(jnp.bitwise_and(u, 1).astype(jnp.float32))

    G = jax.lax.fori_loop(0, 8, cs, jnp.float32(0.0))
    Mincl = (_iota((_L, _L), 0) <= _iota((_L, _L), 1)).astype(jnp.float32)
    Tstrict = (_iota((256, 256), 0) > _iota((256, 256), 1)).astype(jnp.float32)
    rio = _iota((256, _L), 0)
    lio = _iota((256, _L), 1)

    def chunk(t, carry):
        acc, rowc = carry
        u = in_ref[0, pl.ds(t * 256, 256), :]
        fg = jnp.bitwise_and(u, 1).astype(jnp.float32)
        e = jax.lax.bitcast_convert_type(u, jnp.float32)
        rs = jnp.sum(fg, axis=1, keepdims=True)
        F = (jnp.dot(Tstrict, rs, preferred_element_type=jnp.float32)
             + rowc
             + jnp.dot(fg, Mincl, preferred_element_type=jnp.float32))
        i1 = ((t * 256 + rio) * _L + lio + 1).astype(jnp.float32)
        jac = 1.0 - (G - F) / (G + i1 - F)
        un = in_ref[0, pl.ds(jnp.minimum(t * 256 + 256, _R - 1), 1), :]
        e_nh = jax.lax.bitcast_convert_type(un, jnp.float32)
        nh00 = jnp.where(t == 7, 0.0, e_nh[:, 0:1])
        dn = jnp.concatenate([e[1:, 0:1], nh00], axis=0)
        shifted = pltpu.roll(e, _L - 1, 1)
        e_next = jnp.where(lio < _L - 1, shifted, dn)
        acc = acc + jnp.sum(jac * (e - e_next))
        return acc, rowc + jnp.sum(rs)

    acc, _ = jax.lax.fori_loop(
        0, 8, chunk, (jnp.float32(0.0), jnp.float32(0.0)))
    loss_ref[0] = jnp.broadcast_to(acc, (8, _L))
    g_ref[0] = jnp.broadcast_to(G, (8, _L))


def kernel(logits, labels):
    xt = jnp.swapaxes(logits, 0, 1).reshape(_C, _R, _L)
    lab3 = labels.astype(jnp.int32).reshape(_R, _L)

    keys = pl.pallas_call(
        _keys_body,
        grid=(8,),
        in_specs=[
            pl.BlockSpec((_C, 256, _L), lambda i: (0, i, 0)),
            pl.BlockSpec((256, _L), lambda i: (i, 0)),
        ],
        out_specs=pl.BlockSpec((_C, 256, _L), lambda i: (0, i, 0)),
        out_shape=jax.ShapeDtypeStruct((_C, _R, _L), jnp.int32),
    )(xt, lab3)

    _, pong = _sc_sort_build()(keys.reshape(_FLAT))
    skeys = pong.reshape(_C, _R, _L)

    loss_pc, g_pc = pl.pallas_call(
        _loss_body,
        grid=(_C,),
        in_specs=[pl.BlockSpec((1, _R, _L), lambda c: (c, 0, 0))],
        out_specs=[
            pl.BlockSpec((1, 8, _L), lambda c: (c, 0, 0)),
            pl.BlockSpec((1, 8, _L), lambda c: (c, 0, 0)),
        ],
        out_shape=[
            jax.ShapeDtypeStruct((_C, 8, _L), jnp.float32),
            jax.ShapeDtypeStruct((_C, 8, _L), jnp.float32),
        ],
    )(skeys)

    lpc = loss_pc[:, 0, 0]
    present = (g_pc[:, 0, 0] > 0).astype(jnp.float32)
    return jnp.sum(lpc * present) / jnp.maximum(jnp.sum(present), 1.0)


# speedup vs baseline: 18.7783x; 18.7783x over previous
import jax
import jax.numpy as jnp
from jax.experimental import pallas as pl
from jax.experimental.pallas import tpu as pltpu

_P, _C = 262144, 21
_R, _L = 2048, 128
_CH = 16


def _iota(shape, dim):
    return jax.lax.broadcasted_iota(jnp.int32, shape, dim)


def _keys_body(xt_ref, lab_ref, out_ref):
    x = xt_ref[...]
    lab = lab_ref[...]
    m = jnp.max(x, axis=0)
    lse = m + jnp.log(jnp.sum(jnp.exp(x - m[None]), axis=0))
    for c in range(_C):
        fg = lab == c
        e = lse - x[c] + fg.astype(jnp.float32)
        bits = jax.lax.bitcast_convert_type(e, jnp.int32)
        out_ref[c] = jnp.bitwise_or(
            jnp.bitwise_and(bits, -2), fg.astype(jnp.int32))


def _ce_sub(X, s, dirm):
    rb = jnp.bitwise_and(_iota(X.shape, 0), s) != 0
    Pv = jnp.where(rb, pltpu.roll(X, s, 0), pltpu.roll(X, X.shape[0] - s, 0))
    take_max = dirm != rb
    return jnp.where(take_max, jnp.maximum(X, Pv), jnp.minimum(X, Pv))


def _ce_lane(X, s, dirm):
    lb = jnp.bitwise_and(_iota(X.shape, 1), s) != 0
    Pv = jnp.where(lb, pltpu.roll(X, s, 1), pltpu.roll(X, X.shape[1] - s, 1))
    take_max = dirm != lb
    return jnp.where(take_max, jnp.maximum(X, Pv), jnp.minimum(X, Pv))


def _sort_body(in_ref, out_ref, S):
    S[...] = in_ref[0]
    riota = _iota((128, _L), 0)
    liota = _iota((128, _L), 1)

    def p0_chunk(c, _):
        X = S[pl.ds(c * 128, 128), :]
        for k in range(1, 8):
            if k < 7:
                dirm = jnp.bitwise_and(riota, 1 << k) == 0
            else:
                dirm = jnp.bitwise_and(c, 1) == 0
            for j in range(k - 1, -1, -1):
                X = _ce_sub(X, 1 << j, dirm)
        S[pl.ds(c * 128, 128), :] = X
        return 0

    jax.lax.fori_loop(0, _CH, p0_chunk, 0, unroll=2)

    for k in range(8, 19):
        size = 1 << k

        if k >= 12:
            def lane_chunk(c, _, k=k):
                X = S[pl.ds(c * 128, 128), :]
                dirm = jnp.bitwise_and(liota, 1 << (k - 11)) == 0
                for j in range(k - 1, 10, -1):
                    X = _ce_lane(X, 1 << (j - 11), dirm)
                S[pl.ds(c * 128, 128), :] = X
                return 0

            jax.lax.fori_loop(0, _CH, lane_chunk, 0, unroll=2)

        jmax = min(k - 1, 10)
        if jmax >= 7:
            def b1_j(a, _, k=k, size=size, jmax=jmax):
                j = jmax - a
                mm = jnp.left_shift(jnp.int32(1), j - 7)

                def b1_q(q, _):
                    lo = jnp.bitwise_or(
                        jnp.left_shift(jnp.bitwise_and(q, -mm), 1),
                        jnp.bitwise_and(q, mm - 1))
                    A = S[pl.ds(lo * 128, 128), :]
                    B = S[pl.ds((lo + mm) * 128, 128), :]
                    i_low = liota * _R + lo * 128 + riota
                    dirm = jnp.bitwise_and(i_low, size) == 0
                    mn = jnp.minimum(A, B)
                    mx = jnp.maximum(A, B)
                    S[pl.ds(lo * 128, 128), :] = jnp.where(dirm, mx, mn)
                    S[pl.ds((lo + mm) * 128, 128), :] = jnp.where(dirm, mn, mx)
                    return 0

                jax.lax.fori_loop(0, 8, b1_q, 0, unroll=2)
                return 0

            jax.lax.fori_loop(0, jmax - 6, b1_j, 0, unroll=False)

        def b2_chunk(c, _, size=size):
            X = S[pl.ds(c * 128, 128), :]
            i0 = liota * _R + c * 128 + riota
            dirm = jnp.bitwise_and(i0, size) == 0
            for j in range(6, -1, -1):
                X = _ce_sub(X, 1 << j, dirm)
            S[pl.ds(c * 128, 128), :] = X
            return 0

        jax.lax.fori_loop(0, _CH, b2_chunk, 0, unroll=2)

    out_ref[0] = S[...]


def _loss_body(in_ref, loss_ref, g_ref):
    def cs(t, acc):
        u = in_ref[0, pl.ds(t * 256, 256), :]
        return acc + jnp.sum(
            jnp.bitwise_and(u, 1).astype(jnp.float32), axis=0, keepdims=True)

    colsum = jax.lax.fori_loop(0, 8, cs, jnp.zeros((1, _L), jnp.float32))
    G = jnp.sum(colsum)
    Mstrict = (_iota((_L, _L), 0) < _iota((_L, _L), 1)).astype(jnp.float32)
    lane_excl = jnp.dot(colsum, Mstrict, preferred_element_type=jnp.float32)
    T = (_iota((256, 256), 0) >= _iota((256, 256), 1)).astype(jnp.float32)

    u0 = in_ref[0, 0:1, :]
    e0 = jax.lax.bitcast_convert_type(u0, jnp.float32)
    head_last = jnp.where(_iota((1, _L), 1) < _L - 1,
                          pltpu.roll(e0, _L - 1, 1), 0.0)
    rio = _iota((256, _L), 0)
    lio = _iota((256, _L), 1)

    def chunk(t, carry):
        acc, rowc = carry
        u = in_ref[0, pl.ds(t * 256, 256), :]
        fg = jnp.bitwise_and(u, 1).astype(jnp.float32)
        e = jax.lax.bitcast_convert_type(u, jnp.float32)
        F = jnp.dot(T, fg, preferred_element_type=jnp.float32) + rowc + lane_excl
        i1 = (lio * _R + t * 256 + rio + 1).astype(jnp.float32)
        jac = 1.0 - (G - F) / (G + i1 - F)
        un = in_ref[0, pl.ds(jnp.minimum(t * 256 + 256, _R - 1), 1), :]
        e_nh = jax.lax.bitcast_convert_type(un, jnp.float32)
        e_nh = jnp.where(t == 7, head_last, e_nh)
        e_next = jnp.concatenate([e[1:], e_nh], axis=0)
        acc = acc + jnp.sum(jac * (e - e_next))
        rowc_new = rowc + jnp.sum(fg, axis=0, keepdims=True)
        return acc, rowc_new

    acc, _ = jax.lax.fori_loop(
        0, 8, chunk, (jnp.float32(0.0), jnp.zeros((1, _L), jnp.float32)))
    loss_ref[0] = jnp.broadcast_to(acc, (8, _L))
    g_ref[0] = jnp.broadcast_to(G, (8, _L))


def kernel(logits, labels):
    xt = jnp.swapaxes(logits, 0, 1).reshape(_C, _R, _L)
    lab3 = labels.astype(jnp.int32).reshape(_R, _L)

    keys = pl.pallas_call(
        _keys_body,
        grid=(8,),
        in_specs=[
            pl.BlockSpec((_C, 256, _L), lambda i: (0, i, 0)),
            pl.BlockSpec((256, _L), lambda i: (i, 0)),
        ],
        out_specs=pl.BlockSpec((_C, 256, _L), lambda i: (0, i, 0)),
        out_shape=jax.ShapeDtypeStruct((_C, _R, _L), jnp.int32),
    )(xt, lab3)

    skeys = pl.pallas_call(
        _sort_body,
        grid=(_C,),
        in_specs=[pl.BlockSpec((1, _R, _L), lambda c: (c, 0, 0))],
        out_specs=pl.BlockSpec((1, _R, _L), lambda c: (c, 0, 0)),
        out_shape=jax.ShapeDtypeStruct((_C, _R, _L), jnp.int32),
        scratch_shapes=[pltpu.VMEM((_R, _L), jnp.int32)],
    )(keys)

    loss_pc, g_pc = pl.pallas_call(
        _loss_body,
        grid=(_C,),
        in_specs=[pl.BlockSpec((1, _R, _L), lambda c: (c, 0, 0))],
        out_specs=[
            pl.BlockSpec((1, 8, _L), lambda c: (c, 0, 0)),
            pl.BlockSpec((1, 8, _L), lambda c: (c, 0, 0)),
        ],
        out_shape=[
            jax.ShapeDtypeStruct((_C, 8, _L), jnp.float32),
            jax.ShapeDtypeStruct((_C, 8, _L), jnp.float32),
        ],
    )(skeys)

    lpc = loss_pc[:, 0, 0]
    present = (g_pc[:, 0, 0] > 0).astype(jnp.float32)
    return jnp.sum(lpc * present) / jnp.maximum(jnp.sum(present), 1.0)


# speedup vs baseline: 19.6555x; 1.0467x over previous
import jax
import jax.numpy as jnp
from jax.experimental import pallas as pl
from jax.experimental.pallas import tpu as pltpu

_P, _C = 262144, 21
_R, _L = 2048, 128
_CH = 16


def _iota(shape, dim):
    return jax.lax.broadcasted_iota(jnp.int32, shape, dim)


def _keys_body(xt_ref, lab_ref, out_ref):
    x = xt_ref[...]
    lab = lab_ref[...]
    m = jnp.max(x, axis=0)
    lse = m + jnp.log(jnp.sum(jnp.exp(x - m[None]), axis=0))
    for c in range(_C):
        fg = lab == c
        e = lse - x[c] + fg.astype(jnp.float32)
        bits = jax.lax.bitcast_convert_type(e, jnp.int32)
        out_ref[c] = jnp.bitwise_or(
            jnp.bitwise_and(bits, -2), fg.astype(jnp.int32))


def _ce_sub(X, s, dirm):
    rb = jnp.bitwise_and(_iota(X.shape, 0), s) != 0
    Pv = jnp.where(rb, pltpu.roll(X, s, 0), pltpu.roll(X, X.shape[0] - s, 0))
    take_max = dirm != rb
    return jnp.where(take_max, jnp.maximum(X, Pv), jnp.minimum(X, Pv))


def _ce_sub2(X, s, dirm):
    if s < 8:
        return _ce_sub(X, s, dirm)
    R, L = X.shape
    G = R // (2 * s)
    X4 = X.reshape(G, 2, s, L)
    a, b = X4[:, 0], X4[:, 1]
    if getattr(dirm, "ndim", 0) >= 2:
        d4 = dirm.reshape(G, 2, s, L)[:, 0]
    else:
        d4 = dirm
    mn = jnp.minimum(a, b)
    mx = jnp.maximum(a, b)
    na = jnp.where(d4, mx, mn)
    nb = jnp.where(d4, mn, mx)
    return jnp.concatenate([na[:, None], nb[:, None]], axis=1).reshape(R, L)


def _ce_lane(X, s, dirm):
    lb = jnp.bitwise_and(_iota(X.shape, 1), s) != 0
    Pv = jnp.where(lb, pltpu.roll(X, s, 1), pltpu.roll(X, X.shape[1] - s, 1))
    take_max = dirm != lb
    return jnp.where(take_max, jnp.maximum(X, Pv), jnp.minimum(X, Pv))


def _sort_body(in_ref, out_ref, S):
    S[...] = in_ref[0]
    riota = _iota((128, _L), 0)
    liota = _iota((128, _L), 1)

    def p0_chunk(c, _):
        X = S[pl.ds(c * 128, 128), :]
        for k in range(1, 8):
            if k < 7:
                dirm = jnp.bitwise_and(riota, 1 << k) == 0
            else:
                dirm = jnp.bitwise_and(c, 1) == 0
            for j in range(k - 1, -1, -1):
                X = _ce_sub2(X, 1 << j, dirm)
        S[pl.ds(c * 128, 128), :] = X
        return 0

    jax.lax.fori_loop(0, _CH, p0_chunk, 0, unroll=2)

    for k in range(8, 19):
        size = 1 << k

        if k >= 12:
            def lane_chunk(c, _, k=k):
                X = S[pl.ds(c * 128, 128), :]
                dirm = jnp.bitwise_and(liota, 1 << (k - 11)) == 0
                for j in range(k - 1, 10, -1):
                    X = _ce_lane(X, 1 << (j - 11), dirm)
                S[pl.ds(c * 128, 128), :] = X
                return 0

            jax.lax.fori_loop(0, _CH, lane_chunk, 0, unroll=2)

        jmax = min(k - 1, 10)
        if jmax >= 7:
            def b1_j(a, _, k=k, size=size, jmax=jmax):
                j = jmax - a
                mm = jnp.left_shift(jnp.int32(1), j - 7)

                def b1_q(q, _):
                    lo = jnp.bitwise_or(
                        jnp.left_shift(jnp.bitwise_and(q, -mm), 1),
                        jnp.bitwise_and(q, mm - 1))
                    A = S[pl.ds(lo * 128, 128), :]
                    B = S[pl.ds((lo + mm) * 128, 128), :]
                    i_low = liota * _R + lo * 128 + riota
                    dirm = jnp.bitwise_and(i_low, size) == 0
                    mn = jnp.minimum(A, B)
                    mx = jnp.maximum(A, B)
                    S[pl.ds(lo * 128, 128), :] = jnp.where(dirm, mx, mn)
                    S[pl.ds((lo + mm) * 128, 128), :] = jnp.where(dirm, mn, mx)
                    return 0

                jax.lax.fori_loop(0, 8, b1_q, 0, unroll=2)
                return 0

            jax.lax.fori_loop(0, jmax - 6, b1_j, 0, unroll=False)

        def b2_chunk(c, _, size=size):
            X = S[pl.ds(c * 128, 128), :]
            i0 = liota * _R + c * 128 + riota
            dirm = jnp.bitwise_and(i0, size) == 0
            for j in range(6, -1, -1):
                X = _ce_sub2(X, 1 << j, dirm)
            S[pl.ds(c * 128, 128), :] = X
            return 0

        jax.lax.fori_loop(0, _CH, b2_chunk, 0, unroll=2)

    out_ref[0] = S[...]


def _loss_body(in_ref, loss_ref, g_ref):
    def cs(t, acc):
        u = in_ref[0, pl.ds(t * 256, 256), :]
        return acc + jnp.sum(
            jnp.bitwise_and(u, 1).astype(jnp.float32), axis=0, keepdims=True)

    colsum = jax.lax.fori_loop(0, 8, cs, jnp.zeros((1, _L), jnp.float32))
    G = jnp.sum(colsum)
    Mstrict = (_iota((_L, _L), 0) < _iota((_L, _L), 1)).astype(jnp.float32)
    lane_excl = jnp.dot(colsum, Mstrict, preferred_element_type=jnp.float32)
    T = (_iota((256, 256), 0) >= _iota((256, 256), 1)).astype(jnp.float32)

    u0 = in_ref[0, 0:1, :]
    e0 = jax.lax.bitcast_convert_type(u0, jnp.float32)
    head_last = jnp.where(_iota((1, _L), 1) < _L - 1,
                          pltpu.roll(e0, _L - 1, 1), 0.0)
    rio = _iota((256, _L), 0)
    lio = _iota((256, _L), 1)

    def chunk(t, carry):
        acc, rowc = carry
        u = in_ref[0, pl.ds(t * 256, 256), :]
        fg = jnp.bitwise_and(u, 1).astype(jnp.float32)
        e = jax.lax.bitcast_convert_type(u, jnp.float32)
        F = jnp.dot(T, fg, preferred_element_type=jnp.float32) + rowc + lane_excl
        i1 = (lio * _R + t * 256 + rio + 1).astype(jnp.float32)
        jac = 1.0 - (G - F) / (G + i1 - F)
        un = in_ref[0, pl.ds(jnp.minimum(t * 256 + 256, _R - 1), 1), :]
        e_nh = jax.lax.bitcast_convert_type(un, jnp.float32)
        e_nh = jnp.where(t == 7, head_last, e_nh)
        e_next = jnp.concatenate([e[1:], e_nh], axis=0)
        acc = acc + jnp.sum(jac * (e - e_next))
        rowc_new = rowc + jnp.sum(fg, axis=0, keepdims=True)
        return acc, rowc_new

    acc, _ = jax.lax.fori_loop(
        0, 8, chunk, (jnp.float32(0.0), jnp.zeros((1, _L), jnp.float32)))
    loss_ref[0] = jnp.broadcast_to(acc, (8, _L))
    g_ref[0] = jnp.broadcast_to(G, (8, _L))


def kernel(logits, labels):
    xt = jnp.swapaxes(logits, 0, 1).reshape(_C, _R, _L)
    lab3 = labels.astype(jnp.int32).reshape(_R, _L)

    keys = pl.pallas_call(
        _keys_body,
        grid=(8,),
        in_specs=[
            pl.BlockSpec((_C, 256, _L), lambda i: (0, i, 0)),
            pl.BlockSpec((256, _L), lambda i: (i, 0)),
        ],
        out_specs=pl.BlockSpec((_C, 256, _L), lambda i: (0, i, 0)),
        out_shape=jax.ShapeDtypeStruct((_C, _R, _L), jnp.int32),
    )(xt, lab3)

    skeys = pl.pallas_call(
        _sort_body,
        grid=(_C,),
        in_specs=[pl.BlockSpec((1, _R, _L), lambda c: (c, 0, 0))],
        out_specs=pl.BlockSpec((1, _R, _L), lambda c: (c, 0, 0)),
        out_shape=jax.ShapeDtypeStruct((_C, _R, _L), jnp.int32),
        scratch_shapes=[pltpu.VMEM((_R, _L), jnp.int32)],
    )(keys)

    loss_pc, g_pc = pl.pallas_call(
        _loss_body,
        grid=(_C,),
        in_specs=[pl.BlockSpec((1, _R, _L), lambda c: (c, 0, 0))],
        out_specs=[
            pl.BlockSpec((1, 8, _L), lambda c: (c, 0, 0)),
            pl.BlockSpec((1, 8, _L), lambda c: (c, 0, 0)),
        ],
        out_shape=[
            jax.ShapeDtypeStruct((_C, 8, _L), jnp.float32),
            jax.ShapeDtypeStruct((_C, 8, _L), jnp.float32),
        ],
    )(skeys)

    lpc = loss_pc[:, 0, 0]
    present = (g_pc[:, 0, 0] > 0).astype(jnp.float32)
    return jnp.sum(lpc * present) / jnp.maximum(jnp.sum(present), 1.0)


# speedup vs baseline: 19.7562x; 1.0051x over previous
import jax
import jax.numpy as jnp
from jax.experimental import pallas as pl
from jax.experimental.pallas import tpu as pltpu

_P, _C = 262144, 21
_R, _L = 2048, 128
_CH = 16


def _iota(shape, dim):
    return jax.lax.broadcasted_iota(jnp.int32, shape, dim)


def _keys_body(xt_ref, lab_ref, out_ref):
    x = xt_ref[...]
    lab = lab_ref[...]
    m = jnp.max(x, axis=0)
    lse = m + jnp.log(jnp.sum(jnp.exp(x - m[None]), axis=0))
    for c in range(_C):
        fg = lab == c
        e = lse - x[c] + fg.astype(jnp.float32)
        bits = jax.lax.bitcast_convert_type(e, jnp.int32)
        out_ref[c] = jnp.bitwise_or(
            jnp.bitwise_and(bits, -2), fg.astype(jnp.int32))


def _ce_sub(X, s, dirm):
    rb = jnp.bitwise_and(_iota(X.shape, 0), s) != 0
    Pv = jnp.where(rb, pltpu.roll(X, s, 0), pltpu.roll(X, X.shape[0] - s, 0))
    take_max = dirm != rb
    return jnp.where(take_max, jnp.maximum(X, Pv), jnp.minimum(X, Pv))


def _ce_sub2(X, s, dirm):
    if s < 8:
        return _ce_sub(X, s, dirm)
    R, L = X.shape
    G = R // (2 * s)
    X4 = X.reshape(G, 2, s, L)
    a, b = X4[:, 0], X4[:, 1]
    if getattr(dirm, "ndim", 0) >= 2:
        d4 = dirm.reshape(G, 2, s, L)[:, 0]
    else:
        d4 = dirm
    mn = jnp.minimum(a, b)
    mx = jnp.maximum(a, b)
    na = jnp.where(d4, mx, mn)
    nb = jnp.where(d4, mn, mx)
    return jnp.concatenate([na[:, None], nb[:, None]], axis=1).reshape(R, L)


def _ce_lane(X, s, dirm):
    lb = jnp.bitwise_and(_iota(X.shape, 1), s) != 0
    Pv = jnp.where(lb, pltpu.roll(X, s, 1), pltpu.roll(X, X.shape[1] - s, 1))
    take_max = dirm != lb
    return jnp.where(take_max, jnp.maximum(X, Pv), jnp.minimum(X, Pv))


def _sort_body(in_ref, out_ref):
    S = out_ref.at[0]
    S[...] = in_ref[0]
    riota = _iota((128, _L), 0)
    liota = _iota((128, _L), 1)

    def p0_chunk(c, _):
        X = S[pl.ds(c * 128, 128), :]
        for k in range(1, 8):
            if k < 7:
                dirm = jnp.bitwise_and(riota, 1 << k) == 0
            else:
                dirm = jnp.bitwise_and(c, 1) == 0
            for j in range(k - 1, -1, -1):
                X = _ce_sub2(X, 1 << j, dirm)
        S[pl.ds(c * 128, 128), :] = X
        return 0

    jax.lax.fori_loop(0, _CH, p0_chunk, 0, unroll=4)

    for k in range(8, 19):
        size = 1 << k

        if k >= 12:
            def lane_chunk(c, _, k=k):
                X = S[pl.ds(c * 128, 128), :]
                dirm = jnp.bitwise_and(liota, 1 << (k - 11)) == 0
                for j in range(k - 1, 10, -1):
                    X = _ce_lane(X, 1 << (j - 11), dirm)
                S[pl.ds(c * 128, 128), :] = X
                return 0

            jax.lax.fori_loop(0, _CH, lane_chunk, 0, unroll=2)

        jmax = min(k - 1, 10)
        if jmax >= 7:
            def b1_j(a, _, k=k, size=size, jmax=jmax):
                j = jmax - a
                mm = jnp.left_shift(jnp.int32(1), j - 7)

                def b1_q(q, _):
                    lo = jnp.bitwise_or(
                        jnp.left_shift(jnp.bitwise_and(q, -mm), 1),
                        jnp.bitwise_and(q, mm - 1))
                    A = S[pl.ds(lo * 128, 128), :]
                    B = S[pl.ds((lo + mm) * 128, 128), :]
                    i_low = liota * _R + lo * 128 + riota
                    dirm = jnp.bitwise_and(i_low, size) == 0
                    mn = jnp.minimum(A, B)
                    mx = jnp.maximum(A, B)
                    S[pl.ds(lo * 128, 128), :] = jnp.where(dirm, mx, mn)
                    S[pl.ds((lo + mm) * 128, 128), :] = jnp.where(dirm, mn, mx)
                    return 0

                jax.lax.fori_loop(0, 8, b1_q, 0, unroll=2)
                return 0

            jax.lax.fori_loop(0, jmax - 6, b1_j, 0, unroll=False)

        def b2_chunk(c, _, size=size):
            X = S[pl.ds(c * 128, 128), :]
            i0 = liota * _R + c * 128 + riota
            dirm = jnp.bitwise_and(i0, size) == 0
            for j in range(6, -1, -1):
                X = _ce_sub2(X, 1 << j, dirm)
            S[pl.ds(c * 128, 128), :] = X
            return 0

        jax.lax.fori_loop(0, _CH, b2_chunk, 0, unroll=4)


def _loss_body(in_ref, loss_ref, g_ref):
    def cs(t, acc):
        u = in_ref[0, pl.ds(t * 256, 256), :]
        return acc + jnp.sum(
            jnp.bitwise_and(u, 1).astype(jnp.float32), axis=0, keepdims=True)

    colsum = jax.lax.fori_loop(0, 8, cs, jnp.zeros((1, _L), jnp.float32))
    G = jnp.sum(colsum)
    Mstrict = (_iota((_L, _L), 0) < _iota((_L, _L), 1)).astype(jnp.float32)
    lane_excl = jnp.dot(colsum, Mstrict, preferred_element_type=jnp.float32)
    T = (_iota((256, 256), 0) >= _iota((256, 256), 1)).astype(jnp.float32)

    u0 = in_ref[0, 0:1, :]
    e0 = jax.lax.bitcast_convert_type(u0, jnp.float32)
    head_last = jnp.where(_iota((1, _L), 1) < _L - 1,
                          pltpu.roll(e0, _L - 1, 1), 0.0)
    rio = _iota((256, _L), 0)
    lio = _iota((256, _L), 1)

    def chunk(t, carry):
        acc, rowc = carry
        u = in_ref[0, pl.ds(t * 256, 256), :]
        fg = jnp.bitwise_and(u, 1).astype(jnp.float32)
        e = jax.lax.bitcast_convert_type(u, jnp.float32)
        F = jnp.dot(T, fg, preferred_element_type=jnp.float32) + rowc + lane_excl
        i1 = (lio * _R + t * 256 + rio + 1).astype(jnp.float32)
        jac = 1.0 - (G - F) / (G + i1 - F)
        un = in_ref[0, pl.ds(jnp.minimum(t * 256 + 256, _R - 1), 1), :]
        e_nh = jax.lax.bitcast_convert_type(un, jnp.float32)
        e_nh = jnp.where(t == 7, head_last, e_nh)
        e_next = jnp.concatenate([e[1:], e_nh], axis=0)
        acc = acc + jnp.sum(jac * (e - e_next))
        rowc_new = rowc + jnp.sum(fg, axis=0, keepdims=True)
        return acc, rowc_new

    acc, _ = jax.lax.fori_loop(
        0, 8, chunk, (jnp.float32(0.0), jnp.zeros((1, _L), jnp.float32)))
    loss_ref[0] = jnp.broadcast_to(acc, (8, _L))
    g_ref[0] = jnp.broadcast_to(G, (8, _L))


def kernel(logits, labels):
    xt = jnp.swapaxes(logits, 0, 1).reshape(_C, _R, _L)
    lab3 = labels.astype(jnp.int32).reshape(_R, _L)

    keys = pl.pallas_call(
        _keys_body,
        grid=(8,),
        in_specs=[
            pl.BlockSpec((_C, 256, _L), lambda i: (0, i, 0)),
            pl.BlockSpec((256, _L), lambda i: (i, 0)),
        ],
        out_specs=pl.BlockSpec((_C, 256, _L), lambda i: (0, i, 0)),
        out_shape=jax.ShapeDtypeStruct((_C, _R, _L), jnp.int32),
    )(xt, lab3)

    skeys = pl.pallas_call(
        _sort_body,
        grid=(_C,),
        in_specs=[pl.BlockSpec((1, _R, _L), lambda c: (c, 0, 0))],
        out_specs=pl.BlockSpec((1, _R, _L), lambda c: (c, 0, 0)),
        out_shape=jax.ShapeDtypeStruct((_C, _R, _L), jnp.int32),
    )(keys)

    loss_pc, g_pc = pl.pallas_call(
        _loss_body,
        grid=(_C,),
        in_specs=[pl.BlockSpec((1, _R, _L), lambda c: (c, 0, 0))],
        out_specs=[
            pl.BlockSpec((1, 8, _L), lambda c: (c, 0, 0)),
            pl.BlockSpec((1, 8, _L), lambda c: (c, 0, 0)),
        ],
        out_shape=[
            jax.ShapeDtypeStruct((_C, 8, _L), jnp.float32),
            jax.ShapeDtypeStruct((_C, 8, _L), jnp.float32),
        ],
    )(skeys)

    lpc = loss_pc[:, 0, 0]
    present = (g_pc[:, 0, 0] > 0).astype(jnp.float32)
    return jnp.sum(lpc * present) / jnp.maximum(jnp.sum(present), 1.0)


# speedup vs baseline: 20.2871x; 1.0269x over previous
import jax
import jax.numpy as jnp
from jax.experimental import pallas as pl
from jax.experimental.pallas import tpu as pltpu

_P, _C = 262144, 21
_R, _L = 2048, 128
_CH = 16


def _iota(shape, dim):
    return jax.lax.broadcasted_iota(jnp.int32, shape, dim)


def _keys_body(xt_ref, lab_ref, out_ref):
    x = xt_ref[...]
    lab = lab_ref[...]
    m = jnp.max(x, axis=0)
    lse = m + jnp.log(jnp.sum(jnp.exp(x - m[None]), axis=0))
    for c in range(_C):
        fg = lab == c
        e = lse - x[c] + fg.astype(jnp.float32)
        bits = jax.lax.bitcast_convert_type(e, jnp.int32)
        out_ref[c] = jnp.bitwise_or(
            jnp.bitwise_and(bits, -2), fg.astype(jnp.int32))


def _ce_sub(X, s, dirm):
    rb = jnp.bitwise_and(_iota(X.shape, 0), s) != 0
    Pv = jnp.where(rb, pltpu.roll(X, s, 0), pltpu.roll(X, X.shape[0] - s, 0))
    take_max = dirm != rb
    return jnp.where(take_max, jnp.maximum(X, Pv), jnp.minimum(X, Pv))


def _ce_sub2(X, s, dirm):
    if s < 8:
        return _ce_sub(X, s, dirm)
    R, L = X.shape
    G = R // (2 * s)
    X4 = X.reshape(G, 2, s, L)
    a, b = X4[:, 0], X4[:, 1]
    if getattr(dirm, "ndim", 0) >= 2:
        d4 = dirm.reshape(G, 2, s, L)[:, 0]
    else:
        d4 = dirm
    mn = jnp.minimum(a, b)
    mx = jnp.maximum(a, b)
    na = jnp.where(d4, mx, mn)
    nb = jnp.where(d4, mn, mx)
    return jnp.concatenate([na[:, None], nb[:, None]], axis=1).reshape(R, L)


def _ce_lane(X, s, dirm):
    lb = jnp.bitwise_and(_iota(X.shape, 1), s) != 0
    Pv = jnp.where(lb, pltpu.roll(X, s, 1), pltpu.roll(X, X.shape[1] - s, 1))
    take_max = dirm != lb
    return jnp.where(take_max, jnp.maximum(X, Pv), jnp.minimum(X, Pv))


def _sort_body(in_ref, out_ref):
    S = out_ref.at[0]
    S[...] = in_ref[0]
    riota = _iota((128, _L), 0)
    liota = _iota((128, _L), 1)

    def p0_chunk(c, _):
        X = S[pl.ds(c * 128, 128), :]
        for k in range(1, 8):
            if k < 7:
                dirm = jnp.bitwise_and(riota, 1 << k) == 0
            else:
                dirm = jnp.bitwise_and(c, 1) == 0
            for j in range(k - 1, -1, -1):
                X = _ce_sub2(X, 1 << j, dirm)
        S[pl.ds(c * 128, 128), :] = X
        return 0

    jax.lax.fori_loop(0, _CH, p0_chunk, 0, unroll=4)

    for k in range(8, 19):
        size = 1 << k

        if k >= 12:
            def lane_chunk(c, _, k=k):
                X = S[pl.ds(c * 128, 128), :]
                dirm = jnp.bitwise_and(liota, 1 << (k - 11)) == 0
                for j in range(k - 1, 10, -1):
                    X = _ce_lane(X, 1 << (j - 11), dirm)
                S[pl.ds(c * 128, 128), :] = X
                return 0

            jax.lax.fori_loop(0, _CH, lane_chunk, 0, unroll=2)

        jmax = min(k - 1, 10)
        if jmax >= 7:
            l8 = _iota((8, _L), 1)
            r8 = _iota((8, _L), 0)

            def b1_group(g, _, k=k, size=size, jmax=jmax, l8=l8, r8=r8):
                xs = [S[pl.ds((g + 16 * m) * 8, 8), :] for m in range(16)]
                for j in range(jmax, 6, -1):
                    sm = 1 << (j - 7)
                    for m in range(16):
                        if m & sm:
                            continue
                        mh = m | sm
                        i_low = l8 * _R + (g + 16 * m) * 8 + r8
                        dirm = jnp.bitwise_and(i_low, size) == 0
                        mn = jnp.minimum(xs[m], xs[mh])
                        mx = jnp.maximum(xs[m], xs[mh])
                        xs[m] = jnp.where(dirm, mx, mn)
                        xs[mh] = jnp.where(dirm, mn, mx)
                for m in range(16):
                    S[pl.ds((g + 16 * m) * 8, 8), :] = xs[m]
                return 0

            jax.lax.fori_loop(0, 16, b1_group, 0, unroll=False)

        def b2_chunk(c, _, size=size):
            X = S[pl.ds(c * 128, 128), :]
            i0 = liota * _R + c * 128 + riota
            dirm = jnp.bitwise_and(i0, size) == 0
            for j in range(6, -1, -1):
                X = _ce_sub2(X, 1 << j, dirm)
            S[pl.ds(c * 128, 128), :] = X
            return 0

        jax.lax.fori_loop(0, _CH, b2_chunk, 0, unroll=4)


def _loss_body(in_ref, loss_ref, g_ref):
    def cs(t, acc):
        u = in_ref[0, pl.ds(t * 256, 256), :]
        return acc + jnp.sum(
            jnp.bitwise_and(u, 1).astype(jnp.float32), axis=0, keepdims=True)

    colsum = jax.lax.fori_loop(0, 8, cs, jnp.zeros((1, _L), jnp.float32))
    G = jnp.sum(colsum)
    Mstrict = (_iota((_L, _L), 0) < _iota((_L, _L), 1)).astype(jnp.float32)
    lane_excl = jnp.dot(colsum, Mstrict, preferred_element_type=jnp.float32)
    T = (_iota((256, 256), 0) >= _iota((256, 256), 1)).astype(jnp.float32)

    u0 = in_ref[0, 0:1, :]
    e0 = jax.lax.bitcast_convert_type(u0, jnp.float32)
    head_last = jnp.where(_iota((1, _L), 1) < _L - 1,
                          pltpu.roll(e0, _L - 1, 1), 0.0)
    rio = _iota((256, _L), 0)
    lio = _iota((256, _L), 1)

    def chunk(t, carry):
        acc, rowc = carry
        u = in_ref[0, pl.ds(t * 256, 256), :]
        fg = jnp.bitwise_and(u, 1).astype(jnp.float32)
        e = jax.lax.bitcast_convert_type(u, jnp.float32)
        F = jnp.dot(T, fg, preferred_element_type=jnp.float32) + rowc + lane_excl
        i1 = (lio * _R + t * 256 + rio + 1).astype(jnp.float32)
        jac = 1.0 - (G - F) / (G + i1 - F)
        un = in_ref[0, pl.ds(jnp.minimum(t * 256 + 256, _R - 1), 1), :]
        e_nh = jax.lax.bitcast_convert_type(un, jnp.float32)
        e_nh = jnp.where(t == 7, head_last, e_nh)
        e_next = jnp.concatenate([e[1:], e_nh], axis=0)
        acc = acc + jnp.sum(jac * (e - e_next))
        rowc_new = rowc + jnp.sum(fg, axis=0, keepdims=True)
        return acc, rowc_new

    acc, _ = jax.lax.fori_loop(
        0, 8, chunk, (jnp.float32(0.0), jnp.zeros((1, _L), jnp.float32)))
    loss_ref[0] = jnp.broadcast_to(acc, (8, _L))
    g_ref[0] = jnp.broadcast_to(G, (8, _L))


def kernel(logits, labels):
    xt = jnp.swapaxes(logits, 0, 1).reshape(_C, _R, _L)
    lab3 = labels.astype(jnp.int32).reshape(_R, _L)

    keys = pl.pallas_call(
        _keys_body,
        grid=(8,),
        in_specs=[
            pl.BlockSpec((_C, 256, _L), lambda i: (0, i, 0)),
            pl.BlockSpec((256, _L), lambda i: (i, 0)),
        ],
        out_specs=pl.BlockSpec((_C, 256, _L), lambda i: (0, i, 0)),
        out_shape=jax.ShapeDtypeStruct((_C, _R, _L), jnp.int32),
    )(xt, lab3)

    skeys = pl.pallas_call(
        _sort_body,
        grid=(_C,),
        in_specs=[pl.BlockSpec((1, _R, _L), lambda c: (c, 0, 0))],
        out_specs=pl.BlockSpec((1, _R, _L), lambda c: (c, 0, 0)),
        out_shape=jax.ShapeDtypeStruct((_C, _R, _L), jnp.int32),
    )(keys)

    loss_pc, g_pc = pl.pallas_call(
        _loss_body,
        grid=(_C,),
        in_specs=[pl.BlockSpec((1, _R, _L), lambda c: (c, 0, 0))],
        out_specs=[
            pl.BlockSpec((1, 8, _L), lambda c: (c, 0, 0)),
            pl.BlockSpec((1, 8, _L), lambda c: (c, 0, 0)),
        ],
        out_shape=[
            jax.ShapeDtypeStruct((_C, 8, _L), jnp.float32),
            jax.ShapeDtypeStruct((_C, 8, _L), jnp.float32),
        ],
    )(skeys)

    lpc = loss_pc[:, 0, 0]
    present = (g_pc[:, 0, 0] > 0).astype(jnp.float32)
    return jnp.sum(lpc * present) / jnp.maximum(jnp.sum(present), 1.0)
